# baseline (speedup 1.0000x reference)
import functools

import jax
import jax.numpy as jnp
from jax import lax
from jax.experimental import pallas as pl
from jax.experimental.pallas import tpu as pltpu
from jax.experimental.pallas import tpu_sc as plsc

N = 100000
E = 1600000
H = 32
OUT = 16
HH = H // 2

NPAD = 100096
RPT = NPAD // 16
CH = 2000
CHW = 800
VECS = CH // 16
VECSW = CHW // 16

NPK = NPAD // 16
NPKV = N // 16
PW = 16 * H
BR = 368
NB = NPK // BR


def _mesh():
    return plsc.VectorSubcoreMesh(core_axis_name="c", subcore_axis_name="s")


def _zero_fill(buf, nvec):
    zeros = jnp.zeros((16,), jnp.float32)

    def body(i, carry):
        buf[pl.ds(i * 16, 16)] = zeros
        return carry

    lax.fori_loop(0, nvec, body, 0)


def _zero_fill2d(buf, nrows):
    zeros = jnp.zeros((16,), jnp.float32)

    def body(i, carry):
        buf[i, :] = zeros
        return carry

    lax.fori_loop(0, nrows, body, 0)


def _zero_stripe(zbuf, acc, stripe, clen):
    nfull = RPT // clen
    rem = RPT - nfull * clen
    for k in range(nfull):
        pltpu.sync_copy(zbuf, acc.at[pl.ds(stripe + k * clen, clen)])
    if rem:
        pltpu.sync_copy(zbuf.at[pl.ds(0, rem)],
                        acc.at[pl.ds(stripe + nfull * clen, rem)])


def _copy_stripe(src, src_base, dst, dst_base, bounce, clen):
    nfull = RPT // clen
    rem = RPT - nfull * clen
    for k in range(nfull):
        pltpu.sync_copy(src.at[pl.ds(src_base + k * clen, clen)], bounce)
        pltpu.sync_copy(bounce, dst.at[pl.ds(dst_base + k * clen, clen)])
    if rem:
        pltpu.sync_copy(src.at[pl.ds(src_base + nfull * clen, rem)],
                        bounce.at[pl.ds(0, rem)])
        pltpu.sync_copy(bounce.at[pl.ds(0, rem)],
                        dst.at[pl.ds(dst_base + nfull * clen, rem)])


@functools.partial(
    pl.kernel,
    out_type=jax.ShapeDtypeStruct((NPAD,), jnp.float32),
    mesh=_mesh(),
    compiler_params=pltpu.CompilerParams(use_tc_tiling_on_sc=False),
    scratch_types=[
        pltpu.VMEM((CH,), jnp.int32),
        pltpu.VMEM((CH,), jnp.float32),
        pltpu.VMEM_SHARED((NPAD,), jnp.float32),
    ],
)
def _sc_hist(dst_hbm, deg_out, dst_v, val_v, acc):
    c = lax.axis_index("c")
    s = lax.axis_index("s")

    @pl.when(c == 0)
    def _():
        stripe = pl.multiple_of(s * RPT, 8)
        _zero_fill(val_v, VECS)
        _zero_stripe(val_v, acc, stripe, CH)
        ones = jnp.ones((16,), jnp.float32)

        def fill(i, carry):
            val_v[pl.ds(i * 16, 16)] = ones
            return carry

        lax.fori_loop(0, VECS, fill, 0)
        plsc.subcore_barrier()

        ept = E // 16

        def chunk(j, carry):
            base = pl.multiple_of(s * ept + j * CH, 8)
            pltpu.sync_copy(dst_hbm.at[pl.ds(base, CH)], dst_v)
            pltpu.sync_copy(val_v, acc.at[dst_v], add=True)
            return carry

        lax.fori_loop(0, ept // CH, chunk, 0)
        plsc.subcore_barrier()
        _copy_stripe(acc, stripe, deg_out, stripe, val_v, CH)


@functools.partial(
    pl.kernel,
    out_type=jax.ShapeDtypeStruct((2 * NPAD,), jnp.float32),
    mesh=_mesh(),
    compiler_params=pltpu.CompilerParams(use_tc_tiling_on_sc=False),
    scratch_types=[
        pltpu.VMEM((CH,), jnp.int32),
        pltpu.VMEM((CH,), jnp.int32),
        pltpu.VMEM((CH,), jnp.float32),
        pltpu.VMEM_SHARED((NPAD,), jnp.float32),
        pltpu.SemaphoreType.DMA,
    ],
)
def _sc_agg0(src_hbm, dst_hbm, deg_hbm, out, src_v, dst_v, val_v, acc, sem):
    c = lax.axis_index("c")
    s = lax.axis_index("s")
    stripe = pl.multiple_of(s * RPT, 8)

    _zero_fill(val_v, VECS)
    _zero_stripe(val_v, acc, stripe, CH)
    plsc.subcore_barrier()

    epw = E // 32
    wid = c * 16 + s

    def chunk(j, carry):
        base = pl.multiple_of(wid * epw + j * CH, 8)
        pltpu.sync_copy(src_hbm.at[pl.ds(base, CH)], src_v)
        pltpu.sync_copy(dst_hbm.at[pl.ds(base, CH)], dst_v)
        pltpu.async_copy(deg_hbm.at[src_v], val_v, sem).wait()
        pltpu.sync_copy(val_v, acc.at[dst_v], add=True)
        return carry

    lax.fori_loop(0, epw // CH, chunk, 0)
    plsc.subcore_barrier()
    _copy_stripe(acc, stripe, out, c * NPAD + stripe, val_v, CH)


@functools.partial(
    pl.kernel,
    out_type=jax.ShapeDtypeStruct((NPAD, H), jnp.float32),
    mesh=_mesh(),
    compiler_params=pltpu.CompilerParams(use_tc_tiling_on_sc=False),
    scratch_types=[
        pltpu.VMEM((CHW,), jnp.int32),
        pltpu.VMEM((CHW,), jnp.int32),
        pltpu.VMEM((CHW,), jnp.int32),
        pltpu.VMEM((CHW, HH), jnp.float32),
        pltpu.VMEM_SHARED((NPAD, HH), jnp.float32),
        pltpu.SemaphoreType.DMA,
    ],
)
def _sc_aggw(h2_hbm, src_hbm, dst_hbm, out, src_v, idx_v, dst_v, rows_v, acc,
             sem):
    c = lax.axis_index("c")
    s = lax.axis_index("s")
    stripe = pl.multiple_of(s * RPT, 8)

    _zero_fill2d(rows_v, CHW)
    _zero_stripe(rows_v, acc, stripe, CHW)
    plsc.subcore_barrier()

    ept = E // 16

    def chunk(j, carry):
        base = pl.multiple_of(s * ept + j * CHW, 8)
        pltpu.sync_copy(src_hbm.at[pl.ds(base, CHW)], src_v)
        pltpu.sync_copy(dst_hbm.at[pl.ds(base, CHW)], dst_v)

        def cidx(i, cc):
            v = src_v[pl.ds(i * 16, 16)]
            idx_v[pl.ds(i * 16, 16)] = v + v + c
            return cc

        lax.fori_loop(0, VECSW, cidx, 0)
        pltpu.async_copy(h2_hbm.at[idx_v], rows_v, sem).wait()
        pltpu.sync_copy(rows_v, acc.at[dst_v], add=True)
        return carry

    lax.fori_loop(0, ept // CHW, chunk, 0)
    plsc.subcore_barrier()

    col = pl.multiple_of(c * HH, 8)
    nfull = RPT // CHW
    rem = RPT - nfull * CHW
    for k in range(nfull):
        pltpu.sync_copy(acc.at[pl.ds(stripe + k * CHW, CHW)], rows_v)
        pltpu.sync_copy(rows_v,
                        out.at[pl.ds(stripe + k * CHW, CHW), pl.ds(col, HH)])
    if rem:
        pltpu.sync_copy(acc.at[pl.ds(stripe + nfull * CHW, rem)],
                        rows_v.at[pl.ds(0, rem)])
        pltpu.sync_copy(
            rows_v.at[pl.ds(0, rem)],
            out.at[pl.ds(stripe + nfull * CHW, rem), pl.ds(col, HH)])


def _bdot(x, w):
    return jnp.dot(x.astype(jnp.bfloat16), w.astype(jnp.bfloat16),
                   preferred_element_type=jnp.float32)

def _expand32(v, fold):
    return lax.dot_general(v, fold, (((1,), (1,)), ((), ())),
                           preferred_element_type=jnp.float32,
                           precision=lax.Precision.HIGHEST)


def _stats_update(b, z2, fold_ref, z_ref, ssum_ref, ssq_ref, a_ref):
    z_ref[...] = z2
    rows = lax.broadcasted_iota(jnp.int32, (BR, PW), 0) + b * BR
    valid = rows < NPKV
    zm = jnp.where(valid, z2, 0.0)
    cs = jnp.sum(zm, axis=0, keepdims=True)

    @pl.when(b == 0)
    def _():
        ssum_ref[...] = jnp.zeros_like(ssum_ref)
        ssq_ref[...] = jnp.zeros_like(ssq_ref)
        a32 = jnp.dot(cs, fold_ref[...], preferred_element_type=jnp.float32,
                      precision=lax.Precision.HIGHEST) / (16 * BR)
        a_ref[...] = _expand32(a32, fold_ref[...])

    aa = a_ref[...]
    dd = z2 - aa
    sq = jnp.where(valid, dd * dd, 0.0)
    ssum_ref[...] += cs
    ssq_ref[...] += jnp.sum(sq, axis=0, keepdims=True)


def _tca_l0_body(deg_ref, p0_ref, p1_ref, ksel_ref, wa_ref, ba_ref, wb_ref,
                 bb_ref, fold_ref, z_ref, ssum_ref, ssq_ref, a_ref):
    b = pl.program_id(0)
    z0 = deg_ref[...] + p0_ref[0] + p1_ref[0]
    z0x = jnp.dot(z0, ksel_ref[...], preferred_element_type=jnp.float32,
                  precision=lax.Precision.HIGHEST)
    z1 = jnp.maximum(z0x * wa_ref[...] + ba_ref[...], 0.0)
    z2 = _bdot(z1, wb_ref[...])
    z2 = z2 + bb_ref[...]
    _stats_update(b, z2, fold_ref, z_ref, ssum_ref, ssq_ref, a_ref)


_tca_l0 = pl.pallas_call(
    _tca_l0_body,
    grid=(NB,),
    in_specs=[
        pl.BlockSpec((BR, 16), lambda b: (b, 0)),
        pl.BlockSpec((1, BR, 16), lambda b: (0, b, 0)),
        pl.BlockSpec((1, BR, 16), lambda b: (1, b, 0)),
        pl.BlockSpec((16, PW), lambda b: (0, 0)),
        pl.BlockSpec((1, PW), lambda b: (0, 0)),
        pl.BlockSpec((1, PW), lambda b: (0, 0)),
        pl.BlockSpec((PW, PW), lambda b: (0, 0)),
        pl.BlockSpec((1, PW), lambda b: (0, 0)),
        pl.BlockSpec((PW, H), lambda b: (0, 0)),
    ],
    out_specs=[
        pl.BlockSpec((BR, PW), lambda b: (b, 0)),
        pl.BlockSpec((1, PW), lambda b: (0, 0)),
        pl.BlockSpec((1, PW), lambda b: (0, 0)),
        pl.BlockSpec((1, PW), lambda b: (0, 0)),
    ],
    out_shape=[
        jax.ShapeDtypeStruct((NPK, PW), jnp.float32),
        jax.ShapeDtypeStruct((1, PW), jnp.float32),
        jax.ShapeDtypeStruct((1, PW), jnp.float32),
        jax.ShapeDtypeStruct((1, PW), jnp.float32),
    ],
)


def _tca_lw_body(h_ref, agg_ref, wa_ref, ba_ref, wb_ref, bb_ref, fold_ref,
                 z_ref, ssum_ref, ssq_ref, a_ref):
    b = pl.program_id(0)
    z = h_ref[...] + agg_ref[...]
    z1 = jnp.maximum(_bdot(z, wa_ref[...]) + ba_ref[...], 0.0)
    z2 = _bdot(z1, wb_ref[...])
    z2 = z2 + bb_ref[...]
    _stats_update(b, z2, fold_ref, z_ref, ssum_ref, ssq_ref, a_ref)


_tca_lw = pl.pallas_call(
    _tca_lw_body,
    grid=(NB,),
    in_specs=[
        pl.BlockSpec((BR, PW), lambda b: (b, 0)),
        pl.BlockSpec((BR, PW), lambda b: (b, 0)),
        pl.BlockSpec((PW, PW), lambda b: (0, 0)),
        pl.BlockSpec((1, PW), lambda b: (0, 0)),
        pl.BlockSpec((PW, PW), lambda b: (0, 0)),
        pl.BlockSpec((1, PW), lambda b: (0, 0)),
        pl.BlockSpec((PW, H), lambda b: (0, 0)),
    ],
    out_specs=[
        pl.BlockSpec((BR, PW), lambda b: (b, 0)),
        pl.BlockSpec((1, PW), lambda b: (0, 0)),
        pl.BlockSpec((1, PW), lambda b: (0, 0)),
        pl.BlockSpec((1, PW), lambda b: (0, 0)),
    ],
    out_shape=[
        jax.ShapeDtypeStruct((NPK, PW), jnp.float32),
        jax.ShapeDtypeStruct((1, PW), jnp.float32),
        jax.ShapeDtypeStruct((1, PW), jnp.float32),
        jax.ShapeDtypeStruct((1, PW), jnp.float32),
    ],
)


def _bn_coeffs(ssum, ssq, a_pk, fold_ref, g_ref, b_ref):
    fold = fold_ref[...]
    mu = jnp.dot(ssum, fold, preferred_element_type=jnp.float32,
                 precision=lax.Precision.HIGHEST) / N
    msq = jnp.dot(ssq, fold, preferred_element_type=jnp.float32,
                  precision=lax.Precision.HIGHEST) / N
    a32 = jnp.dot(a_pk, fold, preferred_element_type=jnp.float32,
                  precision=lax.Precision.HIGHEST) / 16.0
    dmu = mu - a32
    var = msq - dmu * dmu
    den = jnp.sqrt(var + 1e-5)
    return (_expand32(mu, fold), _expand32(den, fold),
            _expand32(g_ref[...], fold), _expand32(b_ref[...], fold))


def _tcb_body(z_ref, ssum_ref, ssq_ref, a_ref, fold_ref, g_ref, b_ref,
              h_ref):
    mu_pk, den_pk, g_pk, b_pk = _bn_coeffs(
        ssum_ref[...], ssq_ref[...], a_ref[...], fold_ref, g_ref, b_ref)
    h_ref[...] = jnp.maximum(
        (z_ref[...] - mu_pk) / den_pk * g_pk + b_pk, 0.0)


_tcb = pl.pallas_call(
    _tcb_body,
    grid=(NB,),
    in_specs=[
        pl.BlockSpec((BR, PW), lambda b: (b, 0)),
        pl.BlockSpec((1, PW), lambda b: (0, 0)),
        pl.BlockSpec((1, PW), lambda b: (0, 0)),
        pl.BlockSpec((1, PW), lambda b: (0, 0)),
        pl.BlockSpec((PW, H), lambda b: (0, 0)),
        pl.BlockSpec((1, H), lambda b: (0, 0)),
        pl.BlockSpec((1, H), lambda b: (0, 0)),
    ],
    out_specs=pl.BlockSpec((BR, PW), lambda b: (b, 0)),
    out_shape=jax.ShapeDtypeStruct((NPK, PW), jnp.float32),
)


def _tcb_fin_body(z_ref, ssum_ref, ssq_ref, a_ref, fold_ref, g_ref, b_ref,
                  kc_ref, cb_ref, out_ref):
    mu_pk, den_pk, g_pk, b_pk = _bn_coeffs(
        ssum_ref[...], ssq_ref[...], a_ref[...], fold_ref, g_ref, b_ref)
    h = jnp.maximum((z_ref[...] - mu_pk) / den_pk * g_pk + b_pk, 0.0)
    out_ref[...] = _bdot(h, kc_ref[...]) + cb_ref[...]


_tcb_fin = pl.pallas_call(
    _tcb_fin_body,
    grid=(NB,),
    in_specs=[
        pl.BlockSpec((BR, PW), lambda b: (b, 0)),
        pl.BlockSpec((1, PW), lambda b: (0, 0)),
        pl.BlockSpec((1, PW), lambda b: (0, 0)),
        pl.BlockSpec((1, PW), lambda b: (0, 0)),
        pl.BlockSpec((PW, H), lambda b: (0, 0)),
        pl.BlockSpec((1, H), lambda b: (0, 0)),
        pl.BlockSpec((1, H), lambda b: (0, 0)),
        pl.BlockSpec((PW, 16 * OUT), lambda b: (0, 0)),
        pl.BlockSpec((1, 16 * OUT), lambda b: (0, 0)),
    ],
    out_specs=pl.BlockSpec((BR, 16 * OUT), lambda b: (b, 0)),
    out_shape=jax.ShapeDtypeStruct((NPK, 16 * OUT), jnp.float32),
)


def kernel(edge_index, w0a, b0a, w0b, b0b, bn_g0, bn_b0, w1a, b1a, w1b, b1b,
           bn_g1, bn_b1, w2a, b2a, w2b, b2b, bn_g2, bn_b2, w3a, b3a, w3b,
           b3b, bn_g3, bn_b3, cnn_w, cnn_b):
    src = edge_index[0]
    dst = edge_index[1]

    eye16 = jnp.eye(16, dtype=jnp.float32)
    fold = jnp.tile(jnp.eye(H, dtype=jnp.float32), (16, 1))

    def big(w):
        return jnp.kron(eye16, w)

    def tile_row(v):
        return jnp.tile(v, 16).reshape(1, -1)

    deg = _sc_hist(dst)
    aggp0 = _sc_agg0(src, dst, deg)

    ksel = jnp.kron(eye16, jnp.ones((1, H), jnp.float32))
    aggp0_pk = aggp0.reshape(2, NPK, 16)
    z, ssum, ssq, a = _tca_l0(
        deg.reshape(NPK, 16), aggp0_pk, aggp0_pk, ksel,
        tile_row(w0a.reshape(-1)), tile_row(b0a), big(w0b), tile_row(b0b),
        fold)
    h = _tcb(z, ssum, ssq, a, fold, bn_g0.reshape(1, H), bn_b0.reshape(1, H))

    for wa, ba, wb, bb, g, b in (
        (w1a, b1a, w1b, b1b, bn_g1, bn_b1),
        (w2a, b2a, w2b, b2b, bn_g2, bn_b2),
    ):
        agg = _sc_aggw(h.reshape(2 * NPAD, HH), src, dst)
        z, ssum, ssq, a = _tca_lw(h, agg.reshape(NPK, PW), big(wa),
                                  tile_row(ba), big(wb), tile_row(bb), fold)
        h = _tcb(z, ssum, ssq, a, fold, g.reshape(1, H), b.reshape(1, H))

    agg = _sc_aggw(h.reshape(2 * NPAD, HH), src, dst)
    z, ssum, ssq, a = _tca_lw(h, agg.reshape(NPK, PW), big(w3a),
                              tile_row(b3a), big(w3b), tile_row(b3b), fold)
    out = _tcb_fin(z, ssum, ssq, a, fold, bn_g3.reshape(1, H),
                   bn_b3.reshape(1, H), big(cnn_w), tile_row(cnn_b))
    return out.reshape(NPAD, OUT)[:N][None]


# speedup vs baseline: 1.3663x; 1.3663x over previous
import functools

import jax
import jax.numpy as jnp
from jax import lax
from jax.experimental import pallas as pl
from jax.experimental.pallas import tpu as pltpu
from jax.experimental.pallas import tpu_sc as plsc

N = 100000
E = 1600000
H = 32
OUT = 16
HH = H // 2

NPAD = 100096
RPT = NPAD // 16
CH = 2000
CHW = 800
VECS = CH // 16
VECSW = CHW // 16

NPK = NPAD // 16
NPKV = N // 16
PW = 16 * H
BR = 368
NB = NPK // BR


def _mesh():
    return plsc.VectorSubcoreMesh(core_axis_name="c", subcore_axis_name="s")


def _zero_fill(buf, nvec):
    zeros = jnp.zeros((16,), jnp.float32)

    def body(i, carry):
        buf[pl.ds(i * 16, 16)] = zeros
        return carry

    lax.fori_loop(0, nvec, body, 0)


def _zero_fill2d(buf, nrows):
    zeros = jnp.zeros((16,), jnp.float32)

    def body(i, carry):
        buf[i, :] = zeros
        return carry

    lax.fori_loop(0, nrows, body, 0)


def _zero_stripe(zbuf, acc, stripe, clen):
    nfull = RPT // clen
    rem = RPT - nfull * clen
    for k in range(nfull):
        pltpu.sync_copy(zbuf, acc.at[pl.ds(stripe + k * clen, clen)])
    if rem:
        pltpu.sync_copy(zbuf.at[pl.ds(0, rem)],
                        acc.at[pl.ds(stripe + nfull * clen, rem)])


def _copy_stripe(src, src_base, dst, dst_base, bounce, clen):
    nfull = RPT // clen
    rem = RPT - nfull * clen
    for k in range(nfull):
        pltpu.sync_copy(src.at[pl.ds(src_base + k * clen, clen)], bounce)
        pltpu.sync_copy(bounce, dst.at[pl.ds(dst_base + k * clen, clen)])
    if rem:
        pltpu.sync_copy(src.at[pl.ds(src_base + nfull * clen, rem)],
                        bounce.at[pl.ds(0, rem)])
        pltpu.sync_copy(bounce.at[pl.ds(0, rem)],
                        dst.at[pl.ds(dst_base + nfull * clen, rem)])


@functools.partial(
    pl.kernel,
    out_type=jax.ShapeDtypeStruct((NPAD,), jnp.float32),
    mesh=_mesh(),
    compiler_params=pltpu.CompilerParams(use_tc_tiling_on_sc=False),
    scratch_types=[
        pltpu.VMEM((CH,), jnp.int32),
        pltpu.VMEM((CH,), jnp.float32),
        pltpu.VMEM_SHARED((NPAD,), jnp.float32),
    ],
)
def _sc_hist(dst_hbm, deg_out, dst_v, val_v, acc):
    c = lax.axis_index("c")
    s = lax.axis_index("s")

    @pl.when(c == 0)
    def _():
        stripe = pl.multiple_of(s * RPT, 8)
        _zero_fill(val_v, VECS)
        _zero_stripe(val_v, acc, stripe, CH)
        ones = jnp.ones((16,), jnp.float32)

        def fill(i, carry):
            val_v[pl.ds(i * 16, 16)] = ones
            return carry

        lax.fori_loop(0, VECS, fill, 0)
        plsc.subcore_barrier()

        ept = E // 16

        def chunk(j, carry):
            base = pl.multiple_of(s * ept + j * CH, 8)
            pltpu.sync_copy(dst_hbm.at[pl.ds(base, CH)], dst_v)
            pltpu.sync_copy(val_v, acc.at[dst_v], add=True)
            return carry

        lax.fori_loop(0, ept // CH, chunk, 0)
        plsc.subcore_barrier()
        _copy_stripe(acc, stripe, deg_out, stripe, val_v, CH)


@functools.partial(
    pl.kernel,
    out_type=jax.ShapeDtypeStruct((2 * NPAD,), jnp.float32),
    mesh=_mesh(),
    compiler_params=pltpu.CompilerParams(use_tc_tiling_on_sc=False),
    scratch_types=[
        pltpu.VMEM((CH,), jnp.int32),
        pltpu.VMEM((CH,), jnp.int32),
        pltpu.VMEM((CH,), jnp.float32),
        pltpu.VMEM_SHARED((NPAD,), jnp.float32),
        pltpu.SemaphoreType.DMA,
    ],
)
def _sc_agg0(src_hbm, dst_hbm, deg_hbm, out, src_v, dst_v, val_v, acc, sem):
    c = lax.axis_index("c")
    s = lax.axis_index("s")
    stripe = pl.multiple_of(s * RPT, 8)

    _zero_fill(val_v, VECS)
    _zero_stripe(val_v, acc, stripe, CH)
    plsc.subcore_barrier()

    epw = E // 32
    wid = c * 16 + s

    def chunk(j, carry):
        base = pl.multiple_of(wid * epw + j * CH, 8)
        pltpu.sync_copy(src_hbm.at[pl.ds(base, CH)], src_v)
        pltpu.sync_copy(dst_hbm.at[pl.ds(base, CH)], dst_v)
        pltpu.async_copy(deg_hbm.at[src_v], val_v, sem).wait()
        pltpu.sync_copy(val_v, acc.at[dst_v], add=True)
        return carry

    lax.fori_loop(0, epw // CH, chunk, 0)
    plsc.subcore_barrier()
    _copy_stripe(acc, stripe, out, c * NPAD + stripe, val_v, CH)


@functools.partial(
    pl.kernel,
    out_type=jax.ShapeDtypeStruct((NPAD, H), jnp.float32),
    mesh=_mesh(),
    compiler_params=pltpu.CompilerParams(use_tc_tiling_on_sc=False),
    scratch_types=[
        pltpu.VMEM((CHW,), jnp.int32),
        pltpu.VMEM((CHW,), jnp.int32),
        pltpu.VMEM((CHW, HH), jnp.float32),
        pltpu.VMEM((CHW,), jnp.int32),
        pltpu.VMEM((CHW,), jnp.int32),
        pltpu.VMEM((CHW, HH), jnp.float32),
        pltpu.VMEM_SHARED((NPAD, HH), jnp.float32),
        pltpu.SemaphoreType.DMA,
    ],
)
def _sc_aggw(h2_hbm, src_hbm, dst_hbm, out, sv_a, dv_a, rv_a, sv_b, dv_b,
             rv_b, acc, sem):
    c = lax.axis_index("c")
    s = lax.axis_index("s")
    stripe = pl.multiple_of(s * RPT, 8)

    _zero_fill2d(rv_a, CHW)
    _zero_stripe(rv_a, acc, stripe, CHW)
    plsc.subcore_barrier()

    ept = E // 16
    nch = ept // CHW

    def load_idx(j, sv, dv):
        base = pl.multiple_of(s * ept + j * CHW, 8)
        pltpu.sync_copy(src_hbm.at[pl.ds(base, CHW)], sv)
        pltpu.sync_copy(dst_hbm.at[pl.ds(base, CHW)], dv)

        def cidx(i, cc):
            v = sv[pl.ds(i * 16, 16)]
            sv[pl.ds(i * 16, 16)] = v + v + c
            return cc

        lax.fori_loop(0, VECSW, cidx, 0)

    load_idx(0, sv_a, dv_a)
    pltpu.async_copy(h2_hbm.at[sv_a], rv_a, sem)

    def pair(t, carry):
        j0 = 2 * t
        load_idx(j0 + 1, sv_b, dv_b)
        pltpu.make_async_copy(h2_hbm.at[sv_a], rv_a, sem).wait()
        pltpu.async_copy(h2_hbm.at[sv_b], rv_b, sem)
        pltpu.sync_copy(rv_a, acc.at[dv_a], add=True)
        load_idx(j0 + 2, sv_a, dv_a)
        pltpu.make_async_copy(h2_hbm.at[sv_b], rv_b, sem).wait()
        pltpu.async_copy(h2_hbm.at[sv_a], rv_a, sem)
        pltpu.sync_copy(rv_b, acc.at[dv_b], add=True)
        return carry

    lax.fori_loop(0, (nch - 1) // 2, pair, 0)
    pltpu.make_async_copy(h2_hbm.at[sv_a], rv_a, sem).wait()
    pltpu.sync_copy(rv_a, acc.at[dv_a], add=True)
    plsc.subcore_barrier()

    col = pl.multiple_of(c * HH, 8)
    nfull = RPT // CHW
    rem = RPT - nfull * CHW
    for k in range(nfull):
        pltpu.sync_copy(acc.at[pl.ds(stripe + k * CHW, CHW)], rv_a)
        pltpu.sync_copy(rv_a,
                        out.at[pl.ds(stripe + k * CHW, CHW), pl.ds(col, HH)])
    if rem:
        pltpu.sync_copy(acc.at[pl.ds(stripe + nfull * CHW, rem)],
                        rv_a.at[pl.ds(0, rem)])
        pltpu.sync_copy(
            rv_a.at[pl.ds(0, rem)],
            out.at[pl.ds(stripe + nfull * CHW, rem), pl.ds(col, HH)])


def _bdot(x, w):
    return jnp.dot(x.astype(jnp.bfloat16), w.astype(jnp.bfloat16),
                   preferred_element_type=jnp.float32)

def _expand32(v, fold):
    return lax.dot_general(v, fold, (((1,), (1,)), ((), ())),
                           preferred_element_type=jnp.float32,
                           precision=lax.Precision.HIGHEST)


def _stats_update(b, z2, fold_ref, z_ref, ssum_ref, ssq_ref, a_ref):
    z_ref[...] = z2
    rows = lax.broadcasted_iota(jnp.int32, (BR, PW), 0) + b * BR
    valid = rows < NPKV
    zm = jnp.where(valid, z2, 0.0)
    cs = jnp.sum(zm, axis=0, keepdims=True)

    @pl.when(b == 0)
    def _():
        ssum_ref[...] = jnp.zeros_like(ssum_ref)
        ssq_ref[...] = jnp.zeros_like(ssq_ref)
        a32 = jnp.dot(cs, fold_ref[...], preferred_element_type=jnp.float32,
                      precision=lax.Precision.HIGHEST) / (16 * BR)
        a_ref[...] = _expand32(a32, fold_ref[...])

    aa = a_ref[...]
    dd = z2 - aa
    sq = jnp.where(valid, dd * dd, 0.0)
    ssum_ref[...] += cs
    ssq_ref[...] += jnp.sum(sq, axis=0, keepdims=True)


def _tca_l0_body(deg_ref, p0_ref, p1_ref, ksel_ref, wa_ref, ba_ref, wb_ref,
                 bb_ref, fold_ref, z_ref, ssum_ref, ssq_ref, a_ref):
    b = pl.program_id(0)
    z0 = deg_ref[...] + p0_ref[0] + p1_ref[0]
    z0x = jnp.dot(z0, ksel_ref[...], preferred_element_type=jnp.float32,
                  precision=lax.Precision.HIGHEST)
    z1 = jnp.maximum(z0x * wa_ref[...] + ba_ref[...], 0.0)
    z2 = _bdot(z1, wb_ref[...])
    z2 = z2 + bb_ref[...]
    _stats_update(b, z2, fold_ref, z_ref, ssum_ref, ssq_ref, a_ref)


_tca_l0 = pl.pallas_call(
    _tca_l0_body,
    grid=(NB,),
    in_specs=[
        pl.BlockSpec((BR, 16), lambda b: (b, 0)),
        pl.BlockSpec((1, BR, 16), lambda b: (0, b, 0)),
        pl.BlockSpec((1, BR, 16), lambda b: (1, b, 0)),
        pl.BlockSpec((16, PW), lambda b: (0, 0)),
        pl.BlockSpec((1, PW), lambda b: (0, 0)),
        pl.BlockSpec((1, PW), lambda b: (0, 0)),
        pl.BlockSpec((PW, PW), lambda b: (0, 0)),
        pl.BlockSpec((1, PW), lambda b: (0, 0)),
        pl.BlockSpec((PW, H), lambda b: (0, 0)),
    ],
    out_specs=[
        pl.BlockSpec((BR, PW), lambda b: (b, 0)),
        pl.BlockSpec((1, PW), lambda b: (0, 0)),
        pl.BlockSpec((1, PW), lambda b: (0, 0)),
        pl.BlockSpec((1, PW), lambda b: (0, 0)),
    ],
    out_shape=[
        jax.ShapeDtypeStruct((NPK, PW), jnp.float32),
        jax.ShapeDtypeStruct((1, PW), jnp.float32),
        jax.ShapeDtypeStruct((1, PW), jnp.float32),
        jax.ShapeDtypeStruct((1, PW), jnp.float32),
    ],
)


def _tca_lw_body(h_ref, agg_ref, wa_ref, ba_ref, wb_ref, bb_ref, fold_ref,
                 z_ref, ssum_ref, ssq_ref, a_ref):
    b = pl.program_id(0)
    z = h_ref[...] + agg_ref[...]
    z1 = jnp.maximum(_bdot(z, wa_ref[...]) + ba_ref[...], 0.0)
    z2 = _bdot(z1, wb_ref[...])
    z2 = z2 + bb_ref[...]
    _stats_update(b, z2, fold_ref, z_ref, ssum_ref, ssq_ref, a_ref)


_tca_lw = pl.pallas_call(
    _tca_lw_body,
    grid=(NB,),
    in_specs=[
        pl.BlockSpec((BR, PW), lambda b: (b, 0)),
        pl.BlockSpec((BR, PW), lambda b: (b, 0)),
        pl.BlockSpec((PW, PW), lambda b: (0, 0)),
        pl.BlockSpec((1, PW), lambda b: (0, 0)),
        pl.BlockSpec((PW, PW), lambda b: (0, 0)),
        pl.BlockSpec((1, PW), lambda b: (0, 0)),
        pl.BlockSpec((PW, H), lambda b: (0, 0)),
    ],
    out_specs=[
        pl.BlockSpec((BR, PW), lambda b: (b, 0)),
        pl.BlockSpec((1, PW), lambda b: (0, 0)),
        pl.BlockSpec((1, PW), lambda b: (0, 0)),
        pl.BlockSpec((1, PW), lambda b: (0, 0)),
    ],
    out_shape=[
        jax.ShapeDtypeStruct((NPK, PW), jnp.float32),
        jax.ShapeDtypeStruct((1, PW), jnp.float32),
        jax.ShapeDtypeStruct((1, PW), jnp.float32),
        jax.ShapeDtypeStruct((1, PW), jnp.float32),
    ],
)


def _bn_coeffs(ssum, ssq, a_pk, fold_ref, g_ref, b_ref):
    fold = fold_ref[...]
    mu = jnp.dot(ssum, fold, preferred_element_type=jnp.float32,
                 precision=lax.Precision.HIGHEST) / N
    msq = jnp.dot(ssq, fold, preferred_element_type=jnp.float32,
                  precision=lax.Precision.HIGHEST) / N
    a32 = jnp.dot(a_pk, fold, preferred_element_type=jnp.float32,
                  precision=lax.Precision.HIGHEST) / 16.0
    dmu = mu - a32
    var = msq - dmu * dmu
    den = jnp.sqrt(var + 1e-5)
    return (_expand32(mu, fold), _expand32(den, fold),
            _expand32(g_ref[...], fold), _expand32(b_ref[...], fold))


def _tcb_body(z_ref, ssum_ref, ssq_ref, a_ref, fold_ref, g_ref, b_ref,
              h_ref):
    mu_pk, den_pk, g_pk, b_pk = _bn_coeffs(
        ssum_ref[...], ssq_ref[...], a_ref[...], fold_ref, g_ref, b_ref)
    h_ref[...] = jnp.maximum(
        (z_ref[...] - mu_pk) / den_pk * g_pk + b_pk, 0.0)


_tcb = pl.pallas_call(
    _tcb_body,
    grid=(NB,),
    in_specs=[
        pl.BlockSpec((BR, PW), lambda b: (b, 0)),
        pl.BlockSpec((1, PW), lambda b: (0, 0)),
        pl.BlockSpec((1, PW), lambda b: (0, 0)),
        pl.BlockSpec((1, PW), lambda b: (0, 0)),
        pl.BlockSpec((PW, H), lambda b: (0, 0)),
        pl.BlockSpec((1, H), lambda b: (0, 0)),
        pl.BlockSpec((1, H), lambda b: (0, 0)),
    ],
    out_specs=pl.BlockSpec((BR, PW), lambda b: (b, 0)),
    out_shape=jax.ShapeDtypeStruct((NPK, PW), jnp.float32),
)


def _tcb_fin_body(z_ref, ssum_ref, ssq_ref, a_ref, fold_ref, g_ref, b_ref,
                  kc_ref, cb_ref, out_ref):
    mu_pk, den_pk, g_pk, b_pk = _bn_coeffs(
        ssum_ref[...], ssq_ref[...], a_ref[...], fold_ref, g_ref, b_ref)
    h = jnp.maximum((z_ref[...] - mu_pk) / den_pk * g_pk + b_pk, 0.0)
    out_ref[...] = _bdot(h, kc_ref[...]) + cb_ref[...]


_tcb_fin = pl.pallas_call(
    _tcb_fin_body,
    grid=(NB,),
    in_specs=[
        pl.BlockSpec((BR, PW), lambda b: (b, 0)),
        pl.BlockSpec((1, PW), lambda b: (0, 0)),
        pl.BlockSpec((1, PW), lambda b: (0, 0)),
        pl.BlockSpec((1, PW), lambda b: (0, 0)),
        pl.BlockSpec((PW, H), lambda b: (0, 0)),
        pl.BlockSpec((1, H), lambda b: (0, 0)),
        pl.BlockSpec((1, H), lambda b: (0, 0)),
        pl.BlockSpec((PW, 16 * OUT), lambda b: (0, 0)),
        pl.BlockSpec((1, 16 * OUT), lambda b: (0, 0)),
    ],
    out_specs=pl.BlockSpec((BR, 16 * OUT), lambda b: (b, 0)),
    out_shape=jax.ShapeDtypeStruct((NPK, 16 * OUT), jnp.float32),
)


def kernel(edge_index, w0a, b0a, w0b, b0b, bn_g0, bn_b0, w1a, b1a, w1b, b1b,
           bn_g1, bn_b1, w2a, b2a, w2b, b2b, bn_g2, bn_b2, w3a, b3a, w3b,
           b3b, bn_g3, bn_b3, cnn_w, cnn_b):
    src = edge_index[0]
    dst = edge_index[1]

    eye16 = jnp.eye(16, dtype=jnp.float32)
    fold = jnp.tile(jnp.eye(H, dtype=jnp.float32), (16, 1))

    def big(w):
        return jnp.kron(eye16, w)

    def tile_row(v):
        return jnp.tile(v, 16).reshape(1, -1)

    deg = _sc_hist(dst)
    aggp0 = _sc_agg0(src, dst, deg)

    ksel = jnp.kron(eye16, jnp.ones((1, H), jnp.float32))
    aggp0_pk = aggp0.reshape(2, NPK, 16)
    z, ssum, ssq, a = _tca_l0(
        deg.reshape(NPK, 16), aggp0_pk, aggp0_pk, ksel,
        tile_row(w0a.reshape(-1)), tile_row(b0a), big(w0b), tile_row(b0b),
        fold)
    h = _tcb(z, ssum, ssq, a, fold, bn_g0.reshape(1, H), bn_b0.reshape(1, H))

    for wa, ba, wb, bb, g, b in (
        (w1a, b1a, w1b, b1b, bn_g1, bn_b1),
        (w2a, b2a, w2b, b2b, bn_g2, bn_b2),
    ):
        agg = _sc_aggw(h.reshape(2 * NPAD, HH), src, dst)
        z, ssum, ssq, a = _tca_lw(h, agg.reshape(NPK, PW), big(wa),
                                  tile_row(ba), big(wb), tile_row(bb), fold)
        h = _tcb(z, ssum, ssq, a, fold, g.reshape(1, H), b.reshape(1, H))

    agg = _sc_aggw(h.reshape(2 * NPAD, HH), src, dst)
    z, ssum, ssq, a = _tca_lw(h, agg.reshape(NPK, PW), big(w3a),
                              tile_row(b3a), big(w3b), tile_row(b3b), fold)
    out = _tcb_fin(z, ssum, ssq, a, fold, bn_g3.reshape(1, H),
                   bn_b3.reshape(1, H), big(cnn_w), tile_row(cnn_b))
    return out.reshape(NPAD, OUT)[:N][None]
